# baseline (speedup 1.0000x reference)
import functools

import jax
import jax.numpy as jnp
from jax import lax
from jax.experimental import pallas as pl
from jax.experimental.pallas import tpu as pltpu
from jax.experimental.pallas import tpu_sc as plsc

HOOK = 42
LANES = 16
BCHUNK = 16
NBUF = 2
N_WORKERS = 32


def _body(n_chunks, hist, ids_hbm, table_hbm, hooked_hbm, out_hbm,
          idx0, idx1, rows0, rows1, hrep, sg0, sg1, sw0, sw1):
  D = table_hbm.shape[1]
  n_cores = 2
  cid = lax.axis_index("c")
  sid = lax.axis_index("s")
  w = sid * n_cores + cid
  b_per_worker = n_chunks * BCHUNK
  n_full = hist // LANES
  offs = [q * LANES for q in range(n_full)]
  if hist % LANES:
    offs.append(hist - LANES)

  pltpu.sync_copy(hooked_hbm.at[0], hrep.at[0])
  hvecs = [hrep[0, pl.ds(c * LANES, LANES)] for c in range(D // LANES)]

  idx = (idx0, idx1)
  rows = (rows0, rows1)
  sg = (sg0, sg1)
  sw = (sw0, sw1)

  def b_base(chunk_i):
    return w * b_per_worker + chunk_i * BCHUNK

  def stage_and_fire(chunk_i, b):
    pltpu.sync_copy(ids_hbm.at[pl.ds(b_base(chunk_i), BCHUNK)], idx[b])
    for bi in range(BCHUNK):
      pltpu.async_copy(table_hbm.at[idx[b].at[bi]],
                       rows[b].at[pl.ds(bi * hist, hist)],
                       sg[b])

  def drain_gathers(b):
    for bi in range(BCHUNK):
      pltpu.make_async_copy(table_hbm.at[idx[b].at[bi]],
                            rows[b].at[pl.ds(bi * hist, hist)],
                            sg[b]).wait()

  def fixup(b):
    acc = jnp.zeros((LANES,), jnp.bool_)
    for bi in range(BCHUNK):
      for off in offs:
        v = idx[b][bi, pl.ds(off, LANES)]
        acc = acc | (v == HOOK)

    @pl.when(plsc.all_reduce_population_count(acc)[0] > 0)
    def _rare():
      def patch(bi, carry):
        for off in offs:
          v = idx[b][bi, pl.ds(off, LANES)]

          @pl.when(plsc.all_reduce_population_count(v == HOOK)[0] > 0)
          def _subvec():
            for l in range(LANES):
              @pl.when(v[l] == HOOK)
              def _lane():
                r = bi * hist + off + l
                for c in range(len(hvecs)):
                  rows[b][r, pl.ds(c * LANES, LANES)] = hvecs[c]
        return carry
      lax.fori_loop(0, BCHUNK, patch, 0)

  def fire_outwrite(chunk_i, b):
    for bi in range(BCHUNK):
      pltpu.async_copy(rows[b].at[pl.ds(bi * hist, hist)],
                       out_hbm.at[b_base(chunk_i) + bi],
                       sw[b])

  def wait_outwrite(b):
    for bi in range(BCHUNK):
      pltpu.make_async_copy(rows[b].at[pl.ds(bi * hist, hist)],
                            out_hbm.at[bi],
                            sw[b]).wait()

  for b in range(NBUF):
    stage_and_fire(b, b)

  def step(s, carry):
    for b in range(NBUF):
      i = s * NBUF + b
      drain_gathers(b)
      fixup(b)
      fire_outwrite(i, b)

      @pl.when(s < n_chunks // NBUF - 1)
      def _prefetch():
        wait_outwrite(b)
        stage_and_fire(i + NBUF, b)
    return carry

  lax.fori_loop(0, n_chunks // NBUF, step, 0)

  for b in range(NBUF):
    wait_outwrite(b)


def kernel(input_ids, main_table, hooked_table):
  B, L = input_ids.shape
  V, D = main_table.shape
  assert B % (N_WORKERS * BCHUNK * NBUF) == 0 and D % LANES == 0
  n_chunks = B // (N_WORKERS * BCHUNK)

  ids = input_ids.astype(jnp.int32)

  mesh = plsc.VectorSubcoreMesh(core_axis_name="c", subcore_axis_name="s")
  run = pl.kernel(
      functools.partial(_body, n_chunks, L),
      out_type=jax.ShapeDtypeStruct((B, L, D), jnp.float32),
      mesh=mesh,
      compiler_params=pltpu.CompilerParams(
          needs_layout_passes=False, use_tc_tiling_on_sc=False),
      scratch_types=[
          pltpu.VMEM((BCHUNK, L), jnp.int32),
          pltpu.VMEM((BCHUNK, L), jnp.int32),
          pltpu.VMEM((BCHUNK * L, D), jnp.float32),
          pltpu.VMEM((BCHUNK * L, D), jnp.float32),
          pltpu.VMEM((1, D), jnp.float32),
          pltpu.SemaphoreType.DMA,
          pltpu.SemaphoreType.DMA,
          pltpu.SemaphoreType.DMA,
          pltpu.SemaphoreType.DMA,
      ],
  )
  return run(ids, main_table, hooked_table.astype(jnp.float32))


# speedup vs baseline: 1.7807x; 1.7807x over previous
import functools

import jax
import jax.numpy as jnp
from jax import lax
from jax.experimental import pallas as pl
from jax.experimental.pallas import tpu as pltpu
from jax.experimental.pallas import tpu_sc as plsc

HOOK = 42
LANES = 16
BCHUNK = 16
NBUF = 2
N_WORKERS = 32


def _body(n_chunks, hist, ids_hbm, table_hbm, hooked_hbm, out_hbm,
          idx0, idx1, rows0, rows1, hrep, sg0, sg1, sw0, sw1):
  D = table_hbm.shape[1]
  n_cores = 2
  cid = lax.axis_index("c")
  sid = lax.axis_index("s")
  w = sid * n_cores + cid
  b_per_worker = n_chunks * BCHUNK
  n_full = hist // LANES
  offs = [q * LANES for q in range(n_full)]
  if hist % LANES:
    offs.append(hist - LANES)

  pltpu.sync_copy(hooked_hbm.at[0], hrep.at[0])
  hvecs = [hrep[0, pl.ds(c * LANES, LANES)] for c in range(D // LANES)]

  idx = (idx0, idx1)
  rows = (rows0, rows1)
  sg = (sg0, sg1)
  sw = (sw0, sw1)

  def b_base(chunk_i):
    return w * b_per_worker + chunk_i * BCHUNK

  def stage_and_fire(chunk_i, b):
    pltpu.sync_copy(ids_hbm.at[pl.ds(b_base(chunk_i), BCHUNK)], idx[b])
    for bi in range(BCHUNK):
      pltpu.async_copy(table_hbm.at[idx[b].at[bi]],
                       rows[b].at[pl.ds(bi * hist, hist)],
                       sg[b])

  def drain_gathers(b):
    for bi in range(BCHUNK):
      pltpu.make_async_copy(table_hbm.at[idx[b].at[bi]],
                            rows[b].at[pl.ds(bi * hist, hist)],
                            sg[b]).wait()

  def fixup(b):
    acc = jnp.zeros((LANES,), jnp.bool_)
    for bi in range(BCHUNK):
      for off in offs:
        v = idx[b][bi, pl.ds(off, LANES)]
        acc = acc | (v == HOOK)

    @pl.when(plsc.all_reduce_population_count(acc)[0] > 0)
    def _rare():
      def patch(bi, carry):
        for off in offs:
          v = idx[b][bi, pl.ds(off, LANES)]

          @pl.when(plsc.all_reduce_population_count(v == HOOK)[0] > 0)
          def _subvec():
            for l in range(LANES):
              @pl.when(v[l] == HOOK)
              def _lane():
                r = bi * hist + off + l
                for c in range(len(hvecs)):
                  rows[b][r, pl.ds(c * LANES, LANES)] = hvecs[c]
        return carry
      lax.fori_loop(0, BCHUNK, patch, 0)

  def fire_outwrite(chunk_i, b):
    for bi in range(BCHUNK):
      pltpu.async_copy(rows[b].at[pl.ds(bi * hist, hist)],
                       out_hbm.at[b_base(chunk_i) + bi, pl.ds(0, hist),
                                  pl.ds(0, D)],
                       sw[b])

  def wait_outwrite(b):
    for bi in range(BCHUNK):
      pltpu.make_async_copy(rows[b].at[pl.ds(bi * hist, hist)],
                            out_hbm.at[bi, pl.ds(0, hist), pl.ds(0, D)],
                            sw[b]).wait()

  for b in range(NBUF):
    stage_and_fire(b, b)

  def step(s, carry):
    for b in range(NBUF):
      i = s * NBUF + b
      drain_gathers(b)
      fixup(b)
      fire_outwrite(i, b)

      @pl.when(s < n_chunks // NBUF - 1)
      def _prefetch():
        wait_outwrite(b)
        stage_and_fire(i + NBUF, b)
    return carry

  lax.fori_loop(0, n_chunks // NBUF, step, 0)

  for b in range(NBUF):
    wait_outwrite(b)


def kernel(input_ids, main_table, hooked_table):
  B, L = input_ids.shape
  V, D = main_table.shape
  assert B % (N_WORKERS * BCHUNK * NBUF) == 0 and D % LANES == 0
  n_chunks = B // (N_WORKERS * BCHUNK)

  ids = input_ids.astype(jnp.int32)
  pad_l = -(-L // 8) * 8
  pad_d = -(-D // 128) * 128

  mesh = plsc.VectorSubcoreMesh(core_axis_name="c", subcore_axis_name="s")
  run = pl.kernel(
      functools.partial(_body, n_chunks, L),
      out_type=jax.ShapeDtypeStruct((B, pad_l, pad_d), jnp.float32),
      mesh=mesh,
      compiler_params=pltpu.CompilerParams(
          needs_layout_passes=False, use_tc_tiling_on_sc=False),
      scratch_types=[
          pltpu.VMEM((BCHUNK, L), jnp.int32),
          pltpu.VMEM((BCHUNK, L), jnp.int32),
          pltpu.VMEM((BCHUNK * L, D), jnp.float32),
          pltpu.VMEM((BCHUNK * L, D), jnp.float32),
          pltpu.VMEM((1, D), jnp.float32),
          pltpu.SemaphoreType.DMA,
          pltpu.SemaphoreType.DMA,
          pltpu.SemaphoreType.DMA,
          pltpu.SemaphoreType.DMA,
      ],
  )
  padded = run(ids, main_table, hooked_table.astype(jnp.float32))
  return padded[:, :L, :D]
